# Initial kernel scaffold; baseline (speedup 1.0000x reference)
import functools

import jax
import jax.numpy as jnp
from jax import lax
from jax.experimental import pallas as pl
from jax.experimental.pallas import tpu as pltpu
from jax.experimental.pallas import tpu_sc as plsc

_N = 10000
_E = 320000
_H = 128
_NC = 2
_NS = 16
_NW = _NC * _NS
_CH = 128
_CPW = 80
_EP = _NW * _CPW * _CH
_NACC = 10016
_RPT = _NACC // _NS
_TRASH = _N + 8
_EPS = 1e-5


def _sc_degree(dst3, zeros16, ones16):
    mesh = plsc.VectorSubcoreMesh(core_axis_name="c", subcore_axis_name="s")

    @functools.partial(
        pl.kernel,
        out_type=jax.ShapeDtypeStruct((_NC, _NACC, 16), jnp.float32),
        mesh=mesh,
        scratch_types=[
            pltpu.VMEM((_CPW, _CH), jnp.int32),
            pltpu.VMEM((_CH, 16), jnp.float32),
            pltpu.VMEM_SHARED((_NACC, 16), jnp.float32),
        ],
    )
    def k(dst_hbm, zeros_hbm, ones_hbm, out_hbm, dst_v, ones_v, acc):
        c = lax.axis_index("c")
        s = lax.axis_index("s")
        wid = c * _NS + s
        pltpu.sync_copy(dst_hbm.at[wid], dst_v)
        pltpu.sync_copy(ones_hbm, ones_v)
        r0 = s * _RPT
        pltpu.sync_copy(zeros_hbm.at[pl.ds(r0, _RPT)], acc.at[pl.ds(r0, _RPT)])
        plsc.subcore_barrier()

        def body(j, carry):
            pltpu.sync_copy(ones_v, acc.at[dst_v.at[j]], add=True)
            return carry

        lax.fori_loop(0, _CPW, body, 0)
        plsc.subcore_barrier()
        pltpu.sync_copy(acc.at[pl.ds(r0, _RPT)], out_hbm.at[c, pl.ds(r0, _RPT)])

    return k(dst3, zeros16, ones16)


def _sc_spmm(table, src3, dst3, zeros128):
    mesh = plsc.VectorSubcoreMesh(core_axis_name="c", subcore_axis_name="s")

    @functools.partial(
        pl.kernel,
        out_type=jax.ShapeDtypeStruct((_NC, _NACC, _H), jnp.float32),
        mesh=mesh,
        scratch_types=[
            pltpu.VMEM((_CPW, _CH), jnp.int32),
            pltpu.VMEM((_CPW, _CH), jnp.int32),
            pltpu.VMEM((_CH, _H), jnp.float32),
            pltpu.VMEM_SHARED((_NACC, _H), jnp.float32),
            pltpu.SemaphoreType.DMA,
        ],
    )
    def k(table_hbm, src_hbm, dst_hbm, zeros_hbm, out_hbm,
          src_v, dst_v, rbuf, acc, sem):
        c = lax.axis_index("c")
        s = lax.axis_index("s")
        wid = c * _NS + s
        pltpu.sync_copy(src_hbm.at[wid], src_v)
        pltpu.sync_copy(dst_hbm.at[wid], dst_v)
        r0 = s * _RPT
        pltpu.sync_copy(zeros_hbm.at[pl.ds(r0, _RPT)], acc.at[pl.ds(r0, _RPT)])
        plsc.subcore_barrier()

        def body(j, carry):
            pltpu.async_copy(table_hbm.at[src_v.at[j]], rbuf, sem).wait()
            pltpu.sync_copy(rbuf, acc.at[dst_v.at[j]], add=True)
            return carry

        lax.fori_loop(0, _CPW, body, 0)
        plsc.subcore_barrier()
        pltpu.sync_copy(acc.at[pl.ds(r0, _RPT)], out_hbm.at[c, pl.ds(r0, _RPT)])

    return k(table, src3, dst3, zeros128)


_R = 2000
_G = _N // _R


def _tc_head(cnt, x, W0):
    def body(cnt_ref, x_ref, w_ref, dinv_ref, hp_ref):
        blk = cnt_ref[...]
        deg = blk[0, :, 0:1] + blk[1, :, 0:1] + 1.0
        dinv = lax.rsqrt(deg)
        dinv_ref[...] = dinv
        hp_ref[...] = jnp.dot(x_ref[...], w_ref[...],
                              preferred_element_type=jnp.float32) * dinv

    return pl.pallas_call(
        body,
        grid=(_G,),
        in_specs=[
            pl.BlockSpec((_NC, _R, 16), lambda i: (0, i, 0)),
            pl.BlockSpec((_R, _H), lambda i: (i, 0)),
            pl.BlockSpec((_H, _H), lambda i: (0, 0)),
        ],
        out_specs=[
            pl.BlockSpec((_R, 1), lambda i: (i, 0)),
            pl.BlockSpec((_R, _H), lambda i: (i, 0)),
        ],
        out_shape=[
            jax.ShapeDtypeStruct((_N, 1), jnp.float32),
            jax.ShapeDtypeStruct((_N, _H), jnp.float32),
        ],
    )(cnt, x, W0)


def _ln_relu(t, g, be):
    mu = jnp.mean(t, axis=-1, keepdims=True)
    var = jnp.mean((t - mu) ** 2, axis=-1, keepdims=True)
    return jnp.maximum((t - mu) * lax.rsqrt(var + _EPS) * g + be, 0.0)


def _tc_mid(sacc, hp, dinv, b, g, be, Wn):
    def body(s_ref, hp_ref, dinv_ref, b_ref, g_ref, be_ref, w_ref, out_ref):
        sblk = s_ref[...]
        dinv = dinv_ref[...]
        t = (sblk[0] + sblk[1] + hp_ref[...]) * dinv + b_ref[...]
        xx = _ln_relu(t, g_ref[...], be_ref[...])
        out_ref[...] = jnp.dot(xx, w_ref[...],
                               preferred_element_type=jnp.float32) * dinv

    return pl.pallas_call(
        body,
        grid=(_G,),
        in_specs=[
            pl.BlockSpec((_NC, _R, _H), lambda i: (0, i, 0)),
            pl.BlockSpec((_R, _H), lambda i: (i, 0)),
            pl.BlockSpec((_R, 1), lambda i: (i, 0)),
            pl.BlockSpec((1, _H), lambda i: (0, 0)),
            pl.BlockSpec((1, _H), lambda i: (0, 0)),
            pl.BlockSpec((1, _H), lambda i: (0, 0)),
            pl.BlockSpec((_H, _H), lambda i: (0, 0)),
        ],
        out_specs=pl.BlockSpec((_R, _H), lambda i: (i, 0)),
        out_shape=jax.ShapeDtypeStruct((_N, _H), jnp.float32),
    )(sacc, hp, dinv, b, g, be, Wn)


def _tc_final(sacc, hp, dinv, b, g, be, fc1_w, fc1_b, fc2_w, fc2_b):
    def body(s_ref, hp_ref, dinv_ref, b_ref, g_ref, be_ref,
             w1_ref, b1_ref, w2_ref, b2_ref, out_ref, acc_ref):
        i = pl.program_id(0)
        sblk = s_ref[...]
        t = (sblk[0] + sblk[1] + hp_ref[...]) * dinv_ref[...] + b_ref[...]
        xx = _ln_relu(t, g_ref[...], be_ref[...])
        ps = jnp.sum(xx, axis=0, keepdims=True)

        @pl.when(i == 0)
        def _():
            acc_ref[...] = ps

        @pl.when(i > 0)
        def _():
            acc_ref[...] = acc_ref[...] + ps

        @pl.when(i == _G - 1)
        def _():
            pooled = acc_ref[...] * (1.0 / _N)
            z = jnp.maximum(
                jnp.dot(pooled, w1_ref[...],
                        preferred_element_type=jnp.float32) + b1_ref[...], 0.0)
            out_ref[...] = jnp.dot(z, w2_ref[...],
                                   preferred_element_type=jnp.float32) + b2_ref[...]

    return pl.pallas_call(
        body,
        grid=(_G,),
        in_specs=[
            pl.BlockSpec((_NC, _R, _H), lambda i: (0, i, 0)),
            pl.BlockSpec((_R, _H), lambda i: (i, 0)),
            pl.BlockSpec((_R, 1), lambda i: (i, 0)),
            pl.BlockSpec((1, _H), lambda i: (0, 0)),
            pl.BlockSpec((1, _H), lambda i: (0, 0)),
            pl.BlockSpec((1, _H), lambda i: (0, 0)),
            pl.BlockSpec((_H, _H // 2), lambda i: (0, 0)),
            pl.BlockSpec((1, _H // 2), lambda i: (0, 0)),
            pl.BlockSpec((_H // 2, 2), lambda i: (0, 0)),
            pl.BlockSpec((1, 2), lambda i: (0, 0)),
        ],
        out_specs=pl.BlockSpec((1, 2), lambda i: (0, 0)),
        out_shape=jax.ShapeDtypeStruct((1, 2), jnp.float32),
        scratch_shapes=[pltpu.VMEM((1, _H), jnp.float32)],
    )(sacc, hp, dinv, b, g, be, fc1_w, fc1_b, fc2_w, fc2_b)


def kernel(x, edge_index, W0, b0, W1, b1, W2, b2,
           g0, be0, g1, be1, g2, be2, fc1_w, fc1_b, fc2_w, fc2_b):
    src = edge_index[0].astype(jnp.int32)
    dst = edge_index[1].astype(jnp.int32)
    pad = _EP - _E
    src3 = jnp.concatenate([src, jnp.zeros((pad,), jnp.int32)]
                           ).reshape(_NW, _CPW, _CH)
    dst3 = jnp.concatenate([dst, jnp.full((pad,), _TRASH, jnp.int32)]
                           ).reshape(_NW, _CPW, _CH)
    zeros128 = jnp.zeros((_NACC, _H), jnp.float32)
    zeros16 = jnp.zeros((_NACC, 16), jnp.float32)
    ones16 = jnp.ones((_CH, 16), jnp.float32)

    b0r, b1r, b2r = (v.reshape(1, _H) for v in (b0, b1, b2))
    g0r, g1r, g2r = (v.reshape(1, _H) for v in (g0, g1, g2))
    be0r, be1r, be2r = (v.reshape(1, _H) for v in (be0, be1, be2))

    cnt = _sc_degree(dst3, zeros16, ones16)
    dinv, hp = _tc_head(cnt, x, W0)
    s = _sc_spmm(hp, src3, dst3, zeros128)
    hp = _tc_mid(s, hp, dinv, b0r, g0r, be0r, W1)
    s = _sc_spmm(hp, src3, dst3, zeros128)
    hp = _tc_mid(s, hp, dinv, b1r, g1r, be1r, W2)
    s = _sc_spmm(hp, src3, dst3, zeros128)
    return _tc_final(s, hp, dinv, b2r, g2r, be2r,
                     fc1_w, fc1_b.reshape(1, _H // 2),
                     fc2_w, fc2_b.reshape(1, 2))


# speedup vs baseline: 7.8057x; 7.8057x over previous
import functools

import jax
import jax.numpy as jnp
from jax import lax
from jax.experimental import pallas as pl
from jax.experimental.pallas import tpu as pltpu
from jax.experimental.pallas import tpu_sc as plsc

_N = 10000
_E = 320000
_H = 128
_NC = 2
_NS = 16
_NW = _NC * _NS
_RNG = 316
_NACC = _NW * _RNG
_LTRASH = _RNG
_ACCR = 320
_CH = 128
_NCH = 97
_CAP = _NCH * _CH
_SCH = 2048
_NSC = 157
_EPAD = _NSC * _SCH
_FARDST = 16000
_EPS = 1e-5


def _sc_bin(srcp, dstp):
    mesh = plsc.VectorSubcoreMesh(core_axis_name="c", subcore_axis_name="s")

    @functools.partial(
        pl.kernel,
        out_type=(
            jax.ShapeDtypeStruct((_NW, _NCH, _CH), jnp.int32),
            jax.ShapeDtypeStruct((_NW, _NCH, _CH), jnp.int32),
            jax.ShapeDtypeStruct((_NW * 16,), jnp.int32),
        ),
        mesh=mesh,
        scratch_types=[
            pltpu.VMEM((_SCH,), jnp.int32),
            pltpu.VMEM((_SCH,), jnp.int32),
            pltpu.VMEM((_NCH, _CH), jnp.int32),
            pltpu.VMEM((_NCH, _CH), jnp.int32),
            pltpu.VMEM((16,), jnp.int32),
        ],
        compiler_params=pltpu.CompilerParams(needs_layout_passes=False),
    )
    def k(src_hbm, dst_hbm,
          sbin_hbm, dbin_hbm, counts_hbm,
          srcb, dstb, sbin_v, dbin_v, cnt_v):
        c = lax.axis_index("c")
        s = lax.axis_index("s")
        wid = c * _NS + s
        lo = wid * _RNG
        base = s * _ACCR

        def scan_chunk(ci, off_v):
            pltpu.sync_copy(src_hbm.at[pl.ds(ci * _SCH, _SCH)], srcb)
            pltpu.sync_copy(dst_hbm.at[pl.ds(ci * _SCH, _SCH)], dstb)

            def step(j, off):
                d = dstb[pl.ds(j * 16, 16)]
                sv = srcb[pl.ds(j * 16, 16)]
                m = (d >= lo) & (d < lo + _RNG)
                pos = off + plsc.cumsum(m.astype(jnp.int32)) - 1
                prow = lax.shift_right_logical(pos, 7)
                pcol = pos & (_CH - 1)
                plsc.store_scatter(sbin_v, [prow, pcol], sv, mask=m)
                plsc.store_scatter(dbin_v, [prow, pcol], d - lo + base, mask=m)
                return off + plsc.all_reduce_population_count(m)

            return lax.fori_loop(0, _SCH // 16, step, off_v)

        off_v = lax.fori_loop(0, _NSC, scan_chunk,
                              jnp.zeros((16,), jnp.int32))

        iota = lax.iota(jnp.int32, 16)
        for kk in range(_CH // 16):
            pos = off_v + iota + (kk * 16)
            prow = lax.shift_right_logical(pos, 7)
            pcol = pos & (_CH - 1)
            plsc.store_scatter(sbin_v, [prow, pcol],
                               jnp.zeros((16,), jnp.int32))
            plsc.store_scatter(dbin_v, [prow, pcol],
                               jnp.full((16,), _LTRASH, jnp.int32) + base)

        cnt_v[...] = off_v
        pltpu.sync_copy(sbin_v, sbin_hbm.at[wid])
        pltpu.sync_copy(dbin_v, dbin_hbm.at[wid])
        pltpu.sync_copy(cnt_v, counts_hbm.at[pl.ds(wid * 16, 16)])

    return k(srcp, dstp)


def _sc_spmm(table, sbin, dbin, counts, zeros128):
    mesh = plsc.VectorSubcoreMesh(core_axis_name="c", subcore_axis_name="s")

    @functools.partial(
        pl.kernel,
        out_type=jax.ShapeDtypeStruct((_NW, _RNG, _H), jnp.float32),
        mesh=mesh,
        scratch_types=[
            pltpu.VMEM((_NCH, _CH), jnp.int32),
            pltpu.VMEM((_NCH, _CH), jnp.int32),
            pltpu.VMEM((16,), jnp.int32),
            pltpu.VMEM((_CH, _H), jnp.float32),
            pltpu.VMEM_SHARED((_NS * _ACCR, _H), jnp.float32),
            pltpu.SemaphoreType.DMA,
        ],
        compiler_params=pltpu.CompilerParams(needs_layout_passes=False),
    )
    def k(table_hbm, sbin_hbm, dbin_hbm, counts_hbm, zeros_hbm, out_hbm,
          sidx, didx, cnt_v, rbuf, acc, sem):
        c = lax.axis_index("c")
        s = lax.axis_index("s")
        wid = c * _NS + s
        base = s * _ACCR
        pltpu.sync_copy(sbin_hbm.at[wid], sidx)
        pltpu.sync_copy(dbin_hbm.at[wid], didx)
        pltpu.sync_copy(counts_hbm.at[pl.ds(wid * 16, 16)], cnt_v)
        pltpu.sync_copy(zeros_hbm, acc.at[pl.ds(base, _ACCR)])
        count = cnt_v[...][0]
        nch = (count + _CH - 1) >> 7

        def body(j, carry):
            pltpu.async_copy(table_hbm.at[sidx.at[j]], rbuf, sem).wait()
            pltpu.sync_copy(rbuf, acc.at[didx.at[j]], add=True)
            return carry

        lax.fori_loop(0, nch, body, 0)
        pltpu.sync_copy(acc.at[pl.ds(base, _RNG)], out_hbm.at[wid])

    return k(table, sbin, dbin, counts, zeros128)


_R = 2000
_G = _N // _R


def _tc_head(cnt, x, W0):
    def body(cnt_ref, x_ref, w_ref, dinv_ref, hp_ref):
        deg = cnt_ref[...][:, 0:1] + 1.0
        dinv = 1.0 / jnp.sqrt(deg)
        dinv_ref[...] = dinv
        hp_ref[...] = jnp.dot(x_ref[...], w_ref[...],
                              preferred_element_type=jnp.float32) * dinv

    return pl.pallas_call(
        body,
        grid=(_G,),
        in_specs=[
            pl.BlockSpec((_R, _H), lambda i: (i, 0)),
            pl.BlockSpec((_R, _H), lambda i: (i, 0)),
            pl.BlockSpec((_H, _H), lambda i: (0, 0)),
        ],
        out_specs=[
            pl.BlockSpec((_R, 1), lambda i: (i, 0)),
            pl.BlockSpec((_R, _H), lambda i: (i, 0)),
        ],
        out_shape=[
            jax.ShapeDtypeStruct((_N, 1), jnp.float32),
            jax.ShapeDtypeStruct((_N, _H), jnp.float32),
        ],
    )(cnt, x, W0)


def _ln_relu(t, g, be):
    mu = jnp.mean(t, axis=-1, keepdims=True)
    var = jnp.mean((t - mu) ** 2, axis=-1, keepdims=True)
    return jnp.maximum((t - mu) / jnp.sqrt(var + _EPS) * g + be, 0.0)


def _tc_mid(sacc, hp, dinv, b, g, be, Wn):
    def body(s_ref, hp_ref, dinv_ref, b_ref, g_ref, be_ref, w_ref, out_ref):
        dinv = dinv_ref[...]
        t = (s_ref[...] + hp_ref[...]) * dinv + b_ref[...]
        xx = _ln_relu(t, g_ref[...], be_ref[...])
        out_ref[...] = jnp.dot(xx, w_ref[...],
                               preferred_element_type=jnp.float32) * dinv

    return pl.pallas_call(
        body,
        grid=(_G,),
        in_specs=[
            pl.BlockSpec((_R, _H), lambda i: (i, 0)),
            pl.BlockSpec((_R, _H), lambda i: (i, 0)),
            pl.BlockSpec((_R, 1), lambda i: (i, 0)),
            pl.BlockSpec((1, _H), lambda i: (0, 0)),
            pl.BlockSpec((1, _H), lambda i: (0, 0)),
            pl.BlockSpec((1, _H), lambda i: (0, 0)),
            pl.BlockSpec((_H, _H), lambda i: (0, 0)),
        ],
        out_specs=pl.BlockSpec((_R, _H), lambda i: (i, 0)),
        out_shape=jax.ShapeDtypeStruct((_N, _H), jnp.float32),
    )(sacc, hp, dinv, b, g, be, Wn)


def _tc_final(sacc, hp, dinv, b, g, be, fc1_w, fc1_b, fc2_w, fc2_b):
    def body(s_ref, hp_ref, dinv_ref, b_ref, g_ref, be_ref,
             w1_ref, b1_ref, w2_ref, b2_ref, out_ref, acc_ref):
        i = pl.program_id(0)
        t = (s_ref[...] + hp_ref[...]) * dinv_ref[...] + b_ref[...]
        xx = _ln_relu(t, g_ref[...], be_ref[...])
        ps = jnp.sum(xx, axis=0, keepdims=True)

        @pl.when(i == 0)
        def _():
            acc_ref[...] = ps

        @pl.when(i > 0)
        def _():
            acc_ref[...] = acc_ref[...] + ps

        @pl.when(i == _G - 1)
        def _():
            pooled = acc_ref[...] * (1.0 / _N)
            z = jnp.maximum(
                jnp.dot(pooled, w1_ref[...],
                        preferred_element_type=jnp.float32) + b1_ref[...], 0.0)
            out_ref[...] = jnp.dot(z, w2_ref[...],
                                   preferred_element_type=jnp.float32) + b2_ref[...]

    return pl.pallas_call(
        body,
        grid=(_G,),
        in_specs=[
            pl.BlockSpec((_R, _H), lambda i: (i, 0)),
            pl.BlockSpec((_R, _H), lambda i: (i, 0)),
            pl.BlockSpec((_R, 1), lambda i: (i, 0)),
            pl.BlockSpec((1, _H), lambda i: (0, 0)),
            pl.BlockSpec((1, _H), lambda i: (0, 0)),
            pl.BlockSpec((1, _H), lambda i: (0, 0)),
            pl.BlockSpec((_H, _H // 2), lambda i: (0, 0)),
            pl.BlockSpec((1, _H // 2), lambda i: (0, 0)),
            pl.BlockSpec((_H // 2, 2), lambda i: (0, 0)),
            pl.BlockSpec((1, 2), lambda i: (0, 0)),
        ],
        out_specs=pl.BlockSpec((1, 2), lambda i: (0, 0)),
        out_shape=jax.ShapeDtypeStruct((1, 2), jnp.float32),
        scratch_shapes=[pltpu.VMEM((1, _H), jnp.float32)],
    )(sacc, hp, dinv, b, g, be, fc1_w, fc1_b, fc2_w, fc2_b)


def kernel(x, edge_index, W0, b0, W1, b1, W2, b2,
           g0, be0, g1, be1, g2, be2, fc1_w, fc1_b, fc2_w, fc2_b):
    src = edge_index[0].astype(jnp.int32)
    dst = edge_index[1].astype(jnp.int32)
    pad = _EPAD - _E
    srcp = jnp.concatenate([src, jnp.zeros((pad,), jnp.int32)])
    dstp = jnp.concatenate([dst, jnp.full((pad,), _FARDST, jnp.int32)])
    zeros128 = jnp.zeros((_ACCR, _H), jnp.float32)

    b0r, b1r, b2r = (v.reshape(1, _H) for v in (b0, b1, b2))
    g0r, g1r, g2r = (v.reshape(1, _H) for v in (g0, g1, g2))
    be0r, be1r, be2r = (v.reshape(1, _H) for v in (be0, be1, be2))

    sbin, dbin, counts = _sc_bin(srcp, dstp)
    ones_nh = jnp.ones((_N, _H), jnp.float32)
    cnt2 = _sc_spmm(ones_nh, sbin, dbin, counts, zeros128).reshape(_NACC, _H)
    dinv, hp = _tc_head(cnt2, x, W0)
    s = _sc_spmm(hp, sbin, dbin, counts, zeros128).reshape(_NACC, _H)
    hp = _tc_mid(s, hp, dinv, b0r, g0r, be0r, W1)
    s = _sc_spmm(hp, sbin, dbin, counts, zeros128).reshape(_NACC, _H)
    hp = _tc_mid(s, hp, dinv, b1r, g1r, be1r, W2)
    s = _sc_spmm(hp, sbin, dbin, counts, zeros128).reshape(_NACC, _H)
    return _tc_final(s, hp, dinv, b2r, g2r, be2r,
                     fc1_w, fc1_b.reshape(1, _H // 2),
                     fc2_w, fc2_b.reshape(1, 2))


# speedup vs baseline: 9.7507x; 1.2492x over previous
import functools

import jax
import jax.numpy as jnp
from jax import lax
from jax.experimental import pallas as pl
from jax.experimental.pallas import tpu as pltpu
from jax.experimental.pallas import tpu_sc as plsc

_N = 10000
_E = 320000
_H = 128
_NC = 2
_NS = 16
_NW = _NC * _NS
_RNG = 316
_NACC = _NW * _RNG
_LTRASH = _RNG
_ACCR = 320
_CH = 128
_NCH = 97
_CAP = _NCH * _CH
_SCH = 2048
_NSC = 157
_EPAD = _NSC * _SCH
_FARDST = 16000
_EPS = 1e-5


def _sc_bin(srcp, dstp):
    mesh = plsc.VectorSubcoreMesh(core_axis_name="c", subcore_axis_name="s")

    @functools.partial(
        pl.kernel,
        out_type=(
            jax.ShapeDtypeStruct((_NW, _NCH, _CH), jnp.int32),
            jax.ShapeDtypeStruct((_NW, _NCH, _CH), jnp.int32),
            jax.ShapeDtypeStruct((_NW * 16,), jnp.int32),
        ),
        mesh=mesh,
        scratch_types=[
            pltpu.VMEM((_SCH,), jnp.int32),
            pltpu.VMEM((_SCH,), jnp.int32),
            pltpu.VMEM((_NCH, _CH), jnp.int32),
            pltpu.VMEM((_NCH, _CH), jnp.int32),
            pltpu.VMEM((16,), jnp.int32),
        ],
        compiler_params=pltpu.CompilerParams(needs_layout_passes=False),
    )
    def k(src_hbm, dst_hbm,
          sbin_hbm, dbin_hbm, counts_hbm,
          srcb, dstb, sbin_v, dbin_v, cnt_v):
        c = lax.axis_index("c")
        s = lax.axis_index("s")
        wid = c * _NS + s
        lo = wid * _RNG
        base = s * _ACCR

        def scan_chunk(ci, off_v):
            pltpu.sync_copy(src_hbm.at[pl.ds(ci * _SCH, _SCH)], srcb)
            pltpu.sync_copy(dst_hbm.at[pl.ds(ci * _SCH, _SCH)], dstb)

            def step(j, off):
                d = dstb[pl.ds(j * 16, 16)]
                sv = srcb[pl.ds(j * 16, 16)]
                m = (d >= lo) & (d < lo + _RNG)
                pos = off + plsc.cumsum(m.astype(jnp.int32)) - 1
                prow = lax.shift_right_logical(pos, 7)
                pcol = pos & (_CH - 1)
                plsc.store_scatter(sbin_v, [prow, pcol], sv, mask=m)
                plsc.store_scatter(dbin_v, [prow, pcol], d - lo + base, mask=m)
                return off + plsc.all_reduce_population_count(m)

            return lax.fori_loop(0, _SCH // 16, step, off_v)

        off_v = lax.fori_loop(0, _NSC, scan_chunk,
                              jnp.zeros((16,), jnp.int32))

        iota = lax.iota(jnp.int32, 16)
        for kk in range(_CH // 16):
            pos = off_v + iota + (kk * 16)
            prow = lax.shift_right_logical(pos, 7)
            pcol = pos & (_CH - 1)
            plsc.store_scatter(sbin_v, [prow, pcol],
                               jnp.zeros((16,), jnp.int32))
            plsc.store_scatter(dbin_v, [prow, pcol],
                               jnp.full((16,), _LTRASH, jnp.int32) + base)

        cnt_v[...] = off_v
        pltpu.sync_copy(sbin_v, sbin_hbm.at[wid])
        pltpu.sync_copy(dbin_v, dbin_hbm.at[wid])
        pltpu.sync_copy(cnt_v, counts_hbm.at[pl.ds(wid * 16, 16)])

    return k(srcp, dstp)


def _sc_degree(dbin, counts, zeros128, ones128):
    mesh = plsc.VectorSubcoreMesh(core_axis_name="c", subcore_axis_name="s")

    @functools.partial(
        pl.kernel,
        out_type=jax.ShapeDtypeStruct((_NW, _RNG, _H), jnp.float32),
        mesh=mesh,
        scratch_types=[
            pltpu.VMEM((_NCH, _CH), jnp.int32),
            pltpu.VMEM((16,), jnp.int32),
            pltpu.VMEM((_CH, _H), jnp.float32),
            pltpu.VMEM_SHARED((_NS * _ACCR, _H), jnp.float32),
        ],
        compiler_params=pltpu.CompilerParams(needs_layout_passes=False),
    )
    def k(dbin_hbm, counts_hbm, zeros_hbm, ones_hbm, out_hbm,
          didx, cnt_v, ones_v, acc):
        c = lax.axis_index("c")
        s = lax.axis_index("s")
        wid = c * _NS + s
        base = s * _ACCR
        pltpu.sync_copy(dbin_hbm.at[wid], didx)
        pltpu.sync_copy(counts_hbm.at[pl.ds(wid * 16, 16)], cnt_v)
        pltpu.sync_copy(zeros_hbm, acc.at[pl.ds(base, _ACCR)])
        pltpu.sync_copy(ones_hbm, ones_v)
        count = cnt_v[...][0]
        nch = (count + _CH - 1) >> 7

        def body(j, carry):
            pltpu.sync_copy(ones_v, acc.at[didx.at[j]], add=True)
            return carry

        lax.fori_loop(0, nch, body, 0)
        pltpu.sync_copy(acc.at[pl.ds(base, _RNG)], out_hbm.at[wid])

    return k(dbin, counts, zeros128, ones128)


def _sc_spmm(table, sbin, dbin, counts, zeros128):
    mesh = plsc.VectorSubcoreMesh(core_axis_name="c", subcore_axis_name="s")

    @functools.partial(
        pl.kernel,
        out_type=jax.ShapeDtypeStruct((_NW, _RNG, _H), jnp.float32),
        mesh=mesh,
        scratch_types=[
            pltpu.VMEM((_NCH, _CH), jnp.int32),
            pltpu.VMEM((_NCH, _CH), jnp.int32),
            pltpu.VMEM((16,), jnp.int32),
            pltpu.VMEM((2, _CH, _H), jnp.float32),
            pltpu.VMEM_SHARED((_NS * _ACCR, _H), jnp.float32),
            pltpu.SemaphoreType.DMA((2,)),
        ],
        compiler_params=pltpu.CompilerParams(needs_layout_passes=False),
    )
    def k(table_hbm, sbin_hbm, dbin_hbm, counts_hbm, zeros_hbm, out_hbm,
          sidx, didx, cnt_v, rbuf, acc, sem):
        c = lax.axis_index("c")
        s = lax.axis_index("s")
        wid = c * _NS + s
        base = s * _ACCR
        pltpu.sync_copy(sbin_hbm.at[wid], sidx)
        pltpu.sync_copy(dbin_hbm.at[wid], didx)
        pltpu.sync_copy(counts_hbm.at[pl.ds(wid * 16, 16)], cnt_v)
        pltpu.sync_copy(zeros_hbm, acc.at[pl.ds(base, _ACCR)])
        count = cnt_v[...][0]
        nch = (count + _CH - 1) >> 7

        @pl.when(nch > 0)
        def _():
            pltpu.async_copy(table_hbm.at[sidx.at[0]], rbuf.at[0], sem.at[0])

        def body(j, carry):
            p = j & 1
            pltpu.make_async_copy(table_hbm.at[sidx.at[j]], rbuf.at[p],
                                  sem.at[p]).wait()

            @pl.when(j + 1 < nch)
            def _():
                pltpu.async_copy(table_hbm.at[sidx.at[j + 1]],
                                 rbuf.at[1 - p], sem.at[1 - p])

            pltpu.sync_copy(rbuf.at[p], acc.at[didx.at[j]], add=True)
            return carry

        lax.fori_loop(0, nch, body, 0)
        pltpu.sync_copy(acc.at[pl.ds(base, _RNG)], out_hbm.at[wid])

    return k(table, sbin, dbin, counts, zeros128)


_R = 2000
_G = _N // _R


def _tc_head(cnt, x, W0):
    def body(cnt_ref, x_ref, w_ref, dinv_ref, hp_ref):
        deg = cnt_ref[...][:, 0:1] + 1.0
        dinv = 1.0 / jnp.sqrt(deg)
        dinv_ref[...] = dinv
        hp_ref[...] = jnp.dot(x_ref[...], w_ref[...],
                              preferred_element_type=jnp.float32) * dinv

    return pl.pallas_call(
        body,
        grid=(_G,),
        in_specs=[
            pl.BlockSpec((_R, _H), lambda i: (i, 0)),
            pl.BlockSpec((_R, _H), lambda i: (i, 0)),
            pl.BlockSpec((_H, _H), lambda i: (0, 0)),
        ],
        out_specs=[
            pl.BlockSpec((_R, 1), lambda i: (i, 0)),
            pl.BlockSpec((_R, _H), lambda i: (i, 0)),
        ],
        out_shape=[
            jax.ShapeDtypeStruct((_N, 1), jnp.float32),
            jax.ShapeDtypeStruct((_N, _H), jnp.float32),
        ],
    )(cnt, x, W0)


def _ln_relu(t, g, be):
    mu = jnp.mean(t, axis=-1, keepdims=True)
    var = jnp.mean((t - mu) ** 2, axis=-1, keepdims=True)
    return jnp.maximum((t - mu) / jnp.sqrt(var + _EPS) * g + be, 0.0)


def _tc_mid(sacc, hp, dinv, b, g, be, Wn):
    def body(s_ref, hp_ref, dinv_ref, b_ref, g_ref, be_ref, w_ref, out_ref):
        dinv = dinv_ref[...]
        t = (s_ref[...] + hp_ref[...]) * dinv + b_ref[...]
        xx = _ln_relu(t, g_ref[...], be_ref[...])
        out_ref[...] = jnp.dot(xx, w_ref[...],
                               preferred_element_type=jnp.float32) * dinv

    return pl.pallas_call(
        body,
        grid=(_G,),
        in_specs=[
            pl.BlockSpec((_R, _H), lambda i: (i, 0)),
            pl.BlockSpec((_R, _H), lambda i: (i, 0)),
            pl.BlockSpec((_R, 1), lambda i: (i, 0)),
            pl.BlockSpec((1, _H), lambda i: (0, 0)),
            pl.BlockSpec((1, _H), lambda i: (0, 0)),
            pl.BlockSpec((1, _H), lambda i: (0, 0)),
            pl.BlockSpec((_H, _H), lambda i: (0, 0)),
        ],
        out_specs=pl.BlockSpec((_R, _H), lambda i: (i, 0)),
        out_shape=jax.ShapeDtypeStruct((_N, _H), jnp.float32),
    )(sacc, hp, dinv, b, g, be, Wn)


def _tc_final(sacc, hp, dinv, b, g, be, fc1_w, fc1_b, fc2_w, fc2_b):
    def body(s_ref, hp_ref, dinv_ref, b_ref, g_ref, be_ref,
             w1_ref, b1_ref, w2_ref, b2_ref, out_ref, acc_ref):
        i = pl.program_id(0)
        t = (s_ref[...] + hp_ref[...]) * dinv_ref[...] + b_ref[...]
        xx = _ln_relu(t, g_ref[...], be_ref[...])
        ps = jnp.sum(xx, axis=0, keepdims=True)

        @pl.when(i == 0)
        def _():
            acc_ref[...] = ps

        @pl.when(i > 0)
        def _():
            acc_ref[...] = acc_ref[...] + ps

        @pl.when(i == _G - 1)
        def _():
            pooled = acc_ref[...] * (1.0 / _N)
            z = jnp.maximum(
                jnp.dot(pooled, w1_ref[...],
                        preferred_element_type=jnp.float32) + b1_ref[...], 0.0)
            out_ref[...] = jnp.dot(z, w2_ref[...],
                                   preferred_element_type=jnp.float32) + b2_ref[...]

    return pl.pallas_call(
        body,
        grid=(_G,),
        in_specs=[
            pl.BlockSpec((_R, _H), lambda i: (i, 0)),
            pl.BlockSpec((_R, _H), lambda i: (i, 0)),
            pl.BlockSpec((_R, 1), lambda i: (i, 0)),
            pl.BlockSpec((1, _H), lambda i: (0, 0)),
            pl.BlockSpec((1, _H), lambda i: (0, 0)),
            pl.BlockSpec((1, _H), lambda i: (0, 0)),
            pl.BlockSpec((_H, _H // 2), lambda i: (0, 0)),
            pl.BlockSpec((1, _H // 2), lambda i: (0, 0)),
            pl.BlockSpec((_H // 2, 2), lambda i: (0, 0)),
            pl.BlockSpec((1, 2), lambda i: (0, 0)),
        ],
        out_specs=pl.BlockSpec((1, 2), lambda i: (0, 0)),
        out_shape=jax.ShapeDtypeStruct((1, 2), jnp.float32),
        scratch_shapes=[pltpu.VMEM((1, _H), jnp.float32)],
    )(sacc, hp, dinv, b, g, be, fc1_w, fc1_b, fc2_w, fc2_b)


def kernel(x, edge_index, W0, b0, W1, b1, W2, b2,
           g0, be0, g1, be1, g2, be2, fc1_w, fc1_b, fc2_w, fc2_b):
    src = edge_index[0].astype(jnp.int32)
    dst = edge_index[1].astype(jnp.int32)
    pad = _EPAD - _E
    srcp = jnp.concatenate([src, jnp.zeros((pad,), jnp.int32)])
    dstp = jnp.concatenate([dst, jnp.full((pad,), _FARDST, jnp.int32)])
    zeros128 = jnp.zeros((_ACCR, _H), jnp.float32)

    b0r, b1r, b2r = (v.reshape(1, _H) for v in (b0, b1, b2))
    g0r, g1r, g2r = (v.reshape(1, _H) for v in (g0, g1, g2))
    be0r, be1r, be2r = (v.reshape(1, _H) for v in (be0, be1, be2))

    sbin, dbin, counts = _sc_bin(srcp, dstp)
    ones128 = jnp.ones((_CH, _H), jnp.float32)
    cnt2 = _sc_degree(dbin, counts, zeros128, ones128).reshape(_NACC, _H)
    dinv, hp = _tc_head(cnt2, x, W0)
    s = _sc_spmm(hp, sbin, dbin, counts, zeros128).reshape(_NACC, _H)
    hp = _tc_mid(s, hp, dinv, b0r, g0r, be0r, W1)
    s = _sc_spmm(hp, sbin, dbin, counts, zeros128).reshape(_NACC, _H)
    hp = _tc_mid(s, hp, dinv, b1r, g1r, be1r, W2)
    s = _sc_spmm(hp, sbin, dbin, counts, zeros128).reshape(_NACC, _H)
    return _tc_final(s, hp, dinv, b2r, g2r, be2r,
                     fc1_w, fc1_b.reshape(1, _H // 2),
                     fc2_w, fc2_b.reshape(1, 2))


# speedup vs baseline: 13.4810x; 1.3826x over previous
import functools

import jax
import jax.numpy as jnp
from jax import lax
from jax.experimental import pallas as pl
from jax.experimental.pallas import tpu as pltpu
from jax.experimental.pallas import tpu_sc as plsc

_N = 10000
_E = 320000
_H = 128
_NC = 2
_NS = 16
_NW = _NC * _NS
_RNG = 316
_NACC = _NW * _RNG
_LTRASH = _RNG
_ACCR = 320
_CH = 128
_NCH = 97
_CAP = _NCH * _CH
_SCH = 2048
_NSC = 157
_EPAD = _NSC * _SCH
_FARDST = 16000
_EPS = 1e-5


def _sc_bin(srcp, dstp):
    mesh = plsc.VectorSubcoreMesh(core_axis_name="c", subcore_axis_name="s")

    @functools.partial(
        pl.kernel,
        out_type=(
            jax.ShapeDtypeStruct((_NW, _NCH, _CH), jnp.int32),
            jax.ShapeDtypeStruct((_NW, _NCH, _CH), jnp.int32),
            jax.ShapeDtypeStruct((_NW * 16,), jnp.int32),
        ),
        mesh=mesh,
        scratch_types=[
            pltpu.VMEM((2, _SCH), jnp.int32),
            pltpu.VMEM((2, _SCH), jnp.int32),
            pltpu.VMEM((_NCH, _CH), jnp.int32),
            pltpu.VMEM((_NCH, _CH), jnp.int32),
            pltpu.VMEM((16,), jnp.int32),
            pltpu.SemaphoreType.DMA((2,)),
            pltpu.SemaphoreType.DMA((2,)),
        ],
        compiler_params=pltpu.CompilerParams(needs_layout_passes=False),
    )
    def k(src_hbm, dst_hbm,
          sbin_hbm, dbin_hbm, counts_hbm,
          srcb, dstb, sbin_v, dbin_v, cnt_v, sems, semd):
        c = lax.axis_index("c")
        s = lax.axis_index("s")
        wid = c * _NS + s
        lo = wid * _RNG
        base = s * _ACCR

        pltpu.async_copy(src_hbm.at[pl.ds(0, _SCH)], srcb.at[0], sems.at[0])
        pltpu.async_copy(dst_hbm.at[pl.ds(0, _SCH)], dstb.at[0], semd.at[0])

        def scan_chunk(ci, off_v):
            p = ci & 1
            pltpu.make_async_copy(src_hbm.at[pl.ds(ci * _SCH, _SCH)],
                                  srcb.at[p], sems.at[p]).wait()
            pltpu.make_async_copy(dst_hbm.at[pl.ds(ci * _SCH, _SCH)],
                                  dstb.at[p], semd.at[p]).wait()

            @pl.when(ci + 1 < _NSC)
            def _():
                nxt = (ci + 1) * _SCH
                pltpu.async_copy(src_hbm.at[pl.ds(nxt, _SCH)],
                                 srcb.at[1 - p], sems.at[1 - p])
                pltpu.async_copy(dst_hbm.at[pl.ds(nxt, _SCH)],
                                 dstb.at[1 - p], semd.at[1 - p])

            @plsc.parallel_loop(0, _SCH // 16, unroll=4, carry=off_v)
            def step(j, off):
                d = dstb[p, pl.ds(j * 16, 16)]
                sv = srcb[p, pl.ds(j * 16, 16)]
                m = (d >= lo) & (d < lo + _RNG)
                pos = off + plsc.cumsum(m.astype(jnp.int32)) - 1
                prow = lax.shift_right_logical(pos, 7)
                pcol = pos & (_CH - 1)
                plsc.store_scatter(sbin_v, [prow, pcol], sv, mask=m)
                plsc.store_scatter(dbin_v, [prow, pcol], d - lo + base, mask=m)
                return off + plsc.all_reduce_population_count(m)

            return step

        off_v = lax.fori_loop(0, _NSC, scan_chunk,
                              jnp.zeros((16,), jnp.int32))

        iota = lax.iota(jnp.int32, 16)
        for kk in range(_CH // 16):
            pos = off_v + iota + (kk * 16)
            prow = lax.shift_right_logical(pos, 7)
            pcol = pos & (_CH - 1)
            plsc.store_scatter(sbin_v, [prow, pcol],
                               jnp.zeros((16,), jnp.int32))
            plsc.store_scatter(dbin_v, [prow, pcol],
                               jnp.full((16,), _LTRASH, jnp.int32) + base)

        cnt_v[...] = off_v
        pltpu.sync_copy(sbin_v, sbin_hbm.at[wid])
        pltpu.sync_copy(dbin_v, dbin_hbm.at[wid])
        pltpu.sync_copy(cnt_v, counts_hbm.at[pl.ds(wid * 16, 16)])

    return k(srcp, dstp)


def _sc_degree(dbin, counts, zeros128, ones128):
    mesh = plsc.VectorSubcoreMesh(core_axis_name="c", subcore_axis_name="s")

    @functools.partial(
        pl.kernel,
        out_type=jax.ShapeDtypeStruct((_NW, _RNG, _H), jnp.float32),
        mesh=mesh,
        scratch_types=[
            pltpu.VMEM((_NCH, _CH), jnp.int32),
            pltpu.VMEM((16,), jnp.int32),
            pltpu.VMEM((_CH, _H), jnp.float32),
            pltpu.VMEM_SHARED((_NS * _ACCR, _H), jnp.float32),
        ],
        compiler_params=pltpu.CompilerParams(needs_layout_passes=False),
    )
    def k(dbin_hbm, counts_hbm, zeros_hbm, ones_hbm, out_hbm,
          didx, cnt_v, ones_v, acc):
        c = lax.axis_index("c")
        s = lax.axis_index("s")
        wid = c * _NS + s
        base = s * _ACCR
        pltpu.sync_copy(dbin_hbm.at[wid], didx)
        pltpu.sync_copy(counts_hbm.at[pl.ds(wid * 16, 16)], cnt_v)
        pltpu.sync_copy(zeros_hbm, acc.at[pl.ds(base, _ACCR)])
        pltpu.sync_copy(ones_hbm, ones_v)
        count = cnt_v[...][0]
        nch = (count + _CH - 1) >> 7

        def body(j, carry):
            pltpu.sync_copy(ones_v, acc.at[didx.at[j]], add=True)
            return carry

        lax.fori_loop(0, nch, body, 0)
        pltpu.sync_copy(acc.at[pl.ds(base, _RNG)], out_hbm.at[wid])

    return k(dbin, counts, zeros128, ones128)


def _sc_spmm(table, sbin, dbin, counts, zeros128):
    mesh = plsc.VectorSubcoreMesh(core_axis_name="c", subcore_axis_name="s")

    @functools.partial(
        pl.kernel,
        out_type=jax.ShapeDtypeStruct((_NW, _RNG, _H), jnp.float32),
        mesh=mesh,
        scratch_types=[
            pltpu.VMEM((_NCH, _CH), jnp.int32),
            pltpu.VMEM((_NCH, _CH), jnp.int32),
            pltpu.VMEM((16,), jnp.int32),
            pltpu.VMEM((2, _CH, _H), jnp.float32),
            pltpu.VMEM_SHARED((_NS * _ACCR, _H), jnp.float32),
            pltpu.SemaphoreType.DMA((2,)),
        ],
        compiler_params=pltpu.CompilerParams(needs_layout_passes=False),
    )
    def k(table_hbm, sbin_hbm, dbin_hbm, counts_hbm, zeros_hbm, out_hbm,
          sidx, didx, cnt_v, rbuf, acc, sem):
        c = lax.axis_index("c")
        s = lax.axis_index("s")
        wid = c * _NS + s
        base = s * _ACCR
        pltpu.sync_copy(sbin_hbm.at[wid], sidx)
        pltpu.sync_copy(dbin_hbm.at[wid], didx)
        pltpu.sync_copy(counts_hbm.at[pl.ds(wid * 16, 16)], cnt_v)
        pltpu.sync_copy(zeros_hbm, acc.at[pl.ds(base, _ACCR)])
        count = cnt_v[...][0]
        nch = (count + _CH - 1) >> 7

        @pl.when(nch > 0)
        def _():
            pltpu.async_copy(table_hbm.at[sidx.at[0]], rbuf.at[0], sem.at[0])

        def body(j, carry):
            p = j & 1
            pltpu.make_async_copy(table_hbm.at[sidx.at[j]], rbuf.at[p],
                                  sem.at[p]).wait()

            @pl.when(j + 1 < nch)
            def _():
                pltpu.async_copy(table_hbm.at[sidx.at[j + 1]],
                                 rbuf.at[1 - p], sem.at[1 - p])

            pltpu.sync_copy(rbuf.at[p], acc.at[didx.at[j]], add=True)
            return carry

        lax.fori_loop(0, nch, body, 0)
        pltpu.sync_copy(acc.at[pl.ds(base, _RNG)], out_hbm.at[wid])

    return k(table, sbin, dbin, counts, zeros128)


_R = 2000
_G = _N // _R


def _tc_head(cnt, x, W0):
    def body(cnt_ref, x_ref, w_ref, dinv_ref, hp_ref):
        deg = cnt_ref[...][:, 0:1] + 1.0
        dinv = 1.0 / jnp.sqrt(deg)
        dinv_ref[...] = dinv
        hp_ref[...] = jnp.dot(x_ref[...], w_ref[...],
                              preferred_element_type=jnp.float32) * dinv

    return pl.pallas_call(
        body,
        grid=(_G,),
        in_specs=[
            pl.BlockSpec((_R, _H), lambda i: (i, 0)),
            pl.BlockSpec((_R, _H), lambda i: (i, 0)),
            pl.BlockSpec((_H, _H), lambda i: (0, 0)),
        ],
        out_specs=[
            pl.BlockSpec((_R, 1), lambda i: (i, 0)),
            pl.BlockSpec((_R, _H), lambda i: (i, 0)),
        ],
        out_shape=[
            jax.ShapeDtypeStruct((_N, 1), jnp.float32),
            jax.ShapeDtypeStruct((_N, _H), jnp.float32),
        ],
    )(cnt, x, W0)


def _ln_relu(t, g, be):
    mu = jnp.mean(t, axis=-1, keepdims=True)
    var = jnp.mean((t - mu) ** 2, axis=-1, keepdims=True)
    return jnp.maximum((t - mu) / jnp.sqrt(var + _EPS) * g + be, 0.0)


def _tc_mid(sacc, hp, dinv, b, g, be, Wn):
    def body(s_ref, hp_ref, dinv_ref, b_ref, g_ref, be_ref, w_ref, out_ref):
        dinv = dinv_ref[...]
        t = (s_ref[...] + hp_ref[...]) * dinv + b_ref[...]
        xx = _ln_relu(t, g_ref[...], be_ref[...])
        out_ref[...] = jnp.dot(xx, w_ref[...],
                               preferred_element_type=jnp.float32) * dinv

    return pl.pallas_call(
        body,
        grid=(_G,),
        in_specs=[
            pl.BlockSpec((_R, _H), lambda i: (i, 0)),
            pl.BlockSpec((_R, _H), lambda i: (i, 0)),
            pl.BlockSpec((_R, 1), lambda i: (i, 0)),
            pl.BlockSpec((1, _H), lambda i: (0, 0)),
            pl.BlockSpec((1, _H), lambda i: (0, 0)),
            pl.BlockSpec((1, _H), lambda i: (0, 0)),
            pl.BlockSpec((_H, _H), lambda i: (0, 0)),
        ],
        out_specs=pl.BlockSpec((_R, _H), lambda i: (i, 0)),
        out_shape=jax.ShapeDtypeStruct((_N, _H), jnp.float32),
    )(sacc, hp, dinv, b, g, be, Wn)


def _tc_final(sacc, hp, dinv, b, g, be, fc1_w, fc1_b, fc2_w, fc2_b):
    def body(s_ref, hp_ref, dinv_ref, b_ref, g_ref, be_ref,
             w1_ref, b1_ref, w2_ref, b2_ref, out_ref, acc_ref):
        i = pl.program_id(0)
        t = (s_ref[...] + hp_ref[...]) * dinv_ref[...] + b_ref[...]
        xx = _ln_relu(t, g_ref[...], be_ref[...])
        ps = jnp.sum(xx, axis=0, keepdims=True)

        @pl.when(i == 0)
        def _():
            acc_ref[...] = ps

        @pl.when(i > 0)
        def _():
            acc_ref[...] = acc_ref[...] + ps

        @pl.when(i == _G - 1)
        def _():
            pooled = acc_ref[...] * (1.0 / _N)
            z = jnp.maximum(
                jnp.dot(pooled, w1_ref[...],
                        preferred_element_type=jnp.float32) + b1_ref[...], 0.0)
            out_ref[...] = jnp.dot(z, w2_ref[...],
                                   preferred_element_type=jnp.float32) + b2_ref[...]

    return pl.pallas_call(
        body,
        grid=(_G,),
        in_specs=[
            pl.BlockSpec((_R, _H), lambda i: (i, 0)),
            pl.BlockSpec((_R, _H), lambda i: (i, 0)),
            pl.BlockSpec((_R, 1), lambda i: (i, 0)),
            pl.BlockSpec((1, _H), lambda i: (0, 0)),
            pl.BlockSpec((1, _H), lambda i: (0, 0)),
            pl.BlockSpec((1, _H), lambda i: (0, 0)),
            pl.BlockSpec((_H, _H // 2), lambda i: (0, 0)),
            pl.BlockSpec((1, _H // 2), lambda i: (0, 0)),
            pl.BlockSpec((_H // 2, 2), lambda i: (0, 0)),
            pl.BlockSpec((1, 2), lambda i: (0, 0)),
        ],
        out_specs=pl.BlockSpec((1, 2), lambda i: (0, 0)),
        out_shape=jax.ShapeDtypeStruct((1, 2), jnp.float32),
        scratch_shapes=[pltpu.VMEM((1, _H), jnp.float32)],
    )(sacc, hp, dinv, b, g, be, fc1_w, fc1_b, fc2_w, fc2_b)


def kernel(x, edge_index, W0, b0, W1, b1, W2, b2,
           g0, be0, g1, be1, g2, be2, fc1_w, fc1_b, fc2_w, fc2_b):
    src = edge_index[0].astype(jnp.int32)
    dst = edge_index[1].astype(jnp.int32)
    pad = _EPAD - _E
    srcp = jnp.concatenate([src, jnp.zeros((pad,), jnp.int32)])
    dstp = jnp.concatenate([dst, jnp.full((pad,), _FARDST, jnp.int32)])
    zeros128 = jnp.zeros((_ACCR, _H), jnp.float32)

    b0r, b1r, b2r = (v.reshape(1, _H) for v in (b0, b1, b2))
    g0r, g1r, g2r = (v.reshape(1, _H) for v in (g0, g1, g2))
    be0r, be1r, be2r = (v.reshape(1, _H) for v in (be0, be1, be2))

    sbin, dbin, counts = _sc_bin(srcp, dstp)
    ones128 = jnp.ones((_CH, _H), jnp.float32)
    cnt2 = _sc_degree(dbin, counts, zeros128, ones128).reshape(_NACC, _H)
    dinv, hp = _tc_head(cnt2, x, W0)
    s = _sc_spmm(hp, sbin, dbin, counts, zeros128).reshape(_NACC, _H)
    hp = _tc_mid(s, hp, dinv, b0r, g0r, be0r, W1)
    s = _sc_spmm(hp, sbin, dbin, counts, zeros128).reshape(_NACC, _H)
    hp = _tc_mid(s, hp, dinv, b1r, g1r, be1r, W2)
    s = _sc_spmm(hp, sbin, dbin, counts, zeros128).reshape(_NACC, _H)
    return _tc_final(s, hp, dinv, b2r, g2r, be2r,
                     fc1_w, fc1_b.reshape(1, _H // 2),
                     fc2_w, fc2_b.reshape(1, 2))
